# baseline (speedup 1.0000x reference)
import functools

import jax
import jax.numpy as jnp
from jax import lax
from jax.experimental import pallas as pl
from jax.experimental.pallas import tpu as pltpu
from jax.experimental.pallas import tpu_sc as plsc

NC = 2
NS = 16
L = 16
NW = NC * NS

BATCH = 16384
EMB = 32
BPW = BATCH // NW
CH = 128


def _mf_body(user_hbm, item_hbm, ue_hbm, ie_hbm, ub_hbm, ib_hbm, out_hbm,
             idx_u, idx_i, urows, irows, ubias, ibias, outv,
             sem_u, sem_i, sem_bu, sem_bi):
    wid = lax.axis_index("s") * NC + lax.axis_index("c")
    base = wid * BPW

    pltpu.sync_copy(user_hbm.at[pl.ds(base, BPW)], idx_u)
    pltpu.sync_copy(item_hbm.at[pl.ds(base, BPW)], idx_i)

    lane = lax.iota(jnp.int32, L)
    zeros = jnp.zeros((L,), jnp.int32)

    for ch in range(BPW // CH):
        cbase = ch * CH

        def fire(g, carry):
            iu = idx_u[pl.ds(cbase + g * L, L)]
            ii = idx_i[pl.ds(cbase + g * L, L)]
            for l in range(L):
                i = g * L + l
                ri = iu[l]
                rj = ii[l]
                pltpu.async_copy(ue_hbm.at[pl.ds(ri, 1), :],
                                 urows.at[pl.ds(i, 1), :], sem_u)
                pltpu.async_copy(ie_hbm.at[pl.ds(rj, 1), :],
                                 irows.at[pl.ds(i, 1), :], sem_i)
                pltpu.async_copy(ub_hbm.at[pl.ds(ri, 1), :],
                                 ubias.at[pl.ds(i, 1), :], sem_bu)
                pltpu.async_copy(ib_hbm.at[pl.ds(rj, 1), :],
                                 ibias.at[pl.ds(i, 1), :], sem_bi)
            return carry

        lax.fori_loop(0, CH // L, fire, 0)

        pltpu.make_async_copy(ue_hbm.at[pl.ds(0, CH), :], urows, sem_u).wait()
        pltpu.make_async_copy(ie_hbm.at[pl.ds(0, CH), :], irows, sem_i).wait()
        pltpu.make_async_copy(ub_hbm.at[pl.ds(0, CH), :], ubias, sem_bu).wait()
        pltpu.make_async_copy(ib_hbm.at[pl.ds(0, CH), :], ibias, sem_bi).wait()

        def g_body(g, carry):
            rows = g * L + lane
            acc = plsc.load_gather(ubias, [rows, zeros])
            acc = acc + plsc.load_gather(ibias, [rows, zeros])
            for d in range(EMB):
                dcol = jnp.full((L,), d, jnp.int32)
                u = plsc.load_gather(urows, [rows, dcol])
                v = plsc.load_gather(irows, [rows, dcol])
                acc = acc + u * v
            outv[pl.ds(cbase + g * L, L)] = acc
            return carry

        lax.fori_loop(0, CH // L, g_body, 0)

    pltpu.sync_copy(outv, out_hbm.at[pl.ds(base, BPW)])


@functools.partial(jax.jit, static_argnums=())
def _mf_call(user, item, user_emb_w, item_emb_w, user_bias_w, item_bias_w):
    mesh = plsc.VectorSubcoreMesh(core_axis_name="c", subcore_axis_name="s")
    run = pl.kernel(
        _mf_body,
        out_type=jax.ShapeDtypeStruct((BATCH,), jnp.float32),
        mesh=mesh,
        compiler_params=pltpu.CompilerParams(needs_layout_passes=False),
        scratch_types=[
            pltpu.VMEM((BPW,), jnp.int32),
            pltpu.VMEM((BPW,), jnp.int32),
            pltpu.VMEM((CH, EMB), jnp.float32),
            pltpu.VMEM((CH, EMB), jnp.float32),
            pltpu.VMEM((CH, 1), jnp.float32),
            pltpu.VMEM((CH, 1), jnp.float32),
            pltpu.VMEM((BPW,), jnp.float32),
            pltpu.SemaphoreType.DMA,
            pltpu.SemaphoreType.DMA,
            pltpu.SemaphoreType.DMA,
            pltpu.SemaphoreType.DMA,
        ],
    )
    return run(user, item, user_emb_w, item_emb_w, user_bias_w, item_bias_w)


def kernel(user, item, user_emb_w, item_emb_w, user_bias_w, item_bias_w):
    user = user.astype(jnp.int32)
    item = item.astype(jnp.int32)
    return _mf_call(user, item, user_emb_w, item_emb_w,
                    user_bias_w, item_bias_w)


# speedup vs baseline: 1.1544x; 1.1544x over previous
import functools

import jax
import jax.numpy as jnp
from jax import lax
from jax.experimental import pallas as pl
from jax.experimental.pallas import tpu as pltpu
from jax.experimental.pallas import tpu_sc as plsc

NC = 2
NS = 16
L = 16
NW = NC * NS

BATCH = 16384
EMB = 32
BPW = BATCH // NW
IC = 128
NCH = BPW // IC
BG = 16


def _mf_body(user_hbm, item_hbm, ue_hbm, ie_hbm, ub_hbm, ib_hbm, out_hbm,
             *scratch):
    idx_u = scratch[0:NCH]
    idx_i = scratch[NCH:2 * NCH]
    idx_u4 = scratch[2 * NCH:3 * NCH]
    idx_i4 = scratch[3 * NCH:4 * NCH]
    urows = scratch[4 * NCH:5 * NCH]
    irows = scratch[5 * NCH:6 * NCH]
    ubias = scratch[6 * NCH:7 * NCH]
    ibias = scratch[7 * NCH:8 * NCH]
    outv = scratch[8 * NCH]
    sem_u, sem_i, sem_bu, sem_bi = scratch[8 * NCH + 1:8 * NCH + 5]

    wid = lax.axis_index("s") * NC + lax.axis_index("c")
    base = wid * BPW

    for j in range(NCH):
        pltpu.sync_copy(user_hbm.at[pl.ds(base + j * IC, IC)], idx_u[j])
        pltpu.sync_copy(item_hbm.at[pl.ds(base + j * IC, IC)], idx_i[j])

    for j in range(NCH):
        def sh_body(k, carry):
            vu = idx_u[j][pl.ds(k * L, L)]
            vi = idx_i[j][pl.ds(k * L, L)]
            idx_u4[j][pl.ds(k * L, L)] = lax.shift_right_logical(vu, 4)
            idx_i4[j][pl.ds(k * L, L)] = lax.shift_right_logical(vi, 4)
            return carry
        lax.fori_loop(0, IC // L, sh_body, 0)

    for j in range(NCH):
        cu = pltpu.make_async_copy(ue_hbm.at[idx_u[j]], urows[j], sem_u)
        ci = pltpu.make_async_copy(ie_hbm.at[idx_i[j]], irows[j], sem_i)
        cbu = pltpu.make_async_copy(ub_hbm.at[idx_u4[j]], ubias[j], sem_bu)
        cbi = pltpu.make_async_copy(ib_hbm.at[idx_i4[j]], ibias[j], sem_bi)
        cu.start()
        ci.start()
        cbu.start()
        cbi.start()
        cu.wait()
        ci.wait()
        cbu.wait()
        cbi.wait()

    lane = lax.iota(jnp.int32, L)

    for j in range(NCH):
        ub_j, ib_j, u_j, i_j = ubias[j], ibias[j], urows[j], irows[j]
        iu_j, ii_j = idx_u[j], idx_i[j]

        def g_body(g, carry):
            rows = g * L + lane
            ucol = iu_j[pl.ds(g * L, L)] & (BG - 1)
            icol = ii_j[pl.ds(g * L, L)] & (BG - 1)
            acc = plsc.load_gather(ub_j, [rows, ucol])
            acc = acc + plsc.load_gather(ib_j, [rows, icol])
            for d in range(EMB):
                dcol = jnp.full((L,), d, jnp.int32)
                u = plsc.load_gather(u_j, [rows, dcol])
                v = plsc.load_gather(i_j, [rows, dcol])
                acc = acc + u * v
            outv[pl.ds(j * IC + g * L, L)] = acc
            return carry

        lax.fori_loop(0, IC // L, g_body, 0)

    pltpu.sync_copy(outv, out_hbm.at[pl.ds(base, BPW)])


@functools.partial(jax.jit, static_argnums=())
def _mf_call(user, item, user_emb_w, item_emb_w, ub16, ib16):
    mesh = plsc.VectorSubcoreMesh(core_axis_name="c", subcore_axis_name="s")
    scratch = (
        [pltpu.VMEM((IC,), jnp.int32)] * (4 * NCH)
        + [pltpu.VMEM((IC, EMB), jnp.float32)] * (2 * NCH)
        + [pltpu.VMEM((IC, BG), jnp.float32)] * (2 * NCH)
        + [pltpu.VMEM((BPW,), jnp.float32)]
        + [pltpu.SemaphoreType.DMA] * 4
    )
    run = pl.kernel(
        _mf_body,
        out_type=jax.ShapeDtypeStruct((BATCH,), jnp.float32),
        mesh=mesh,
        compiler_params=pltpu.CompilerParams(needs_layout_passes=False,
                                             use_tc_tiling_on_sc=False),
        scratch_types=scratch,
    )
    return run(user, item, user_emb_w, item_emb_w, ub16, ib16)


def kernel(user, item, user_emb_w, item_emb_w, user_bias_w, item_bias_w):
    user = user.astype(jnp.int32)
    item = item.astype(jnp.int32)
    ub16 = user_bias_w.reshape(-1, BG)
    ib16 = item_bias_w.reshape(-1, BG)
    return _mf_call(user, item, user_emb_w, item_emb_w, ub16, ib16)
